# Initial kernel scaffold; baseline (speedup 1.0000x reference)
import jax
import jax.numpy as jnp
from jax.experimental import pallas as pl


def kernel(x, item_starts, batch_idx, batch_len, emb_table, W0, b0, W1, b1, Wh, bh):
    raise NotImplementedError("write your pallas kernel here")


# speedup vs baseline: 1.0038x; 1.0038x over previous
import jax
import jax.numpy as jnp
from jax.experimental import pallas as pl

_N = 50000
_B = 2000
_D = 64
_NB = 1000


def _onehot(bidx_block):
    iota = jax.lax.broadcasted_iota(jnp.int32, (bidx_block.shape[0], _B), 1)
    return (bidx_block == iota).astype(jnp.float32)


def _segsum_kernel(emb_ref, bidx_ref, m_ref):
    step = pl.program_id(0)

    @pl.when(step == 0)
    def _():
        m_ref[...] = jnp.zeros_like(m_ref)

    oh = _onehot(bidx_ref[...])
    m_ref[...] += jax.lax.dot_general(
        oh, emb_ref[...], (((0,), (0,)), ((), ())),
        preferred_element_type=jnp.float32)


def _segsum(emb, bidx2d):
    return pl.pallas_call(
        _segsum_kernel,
        grid=(_N // _NB,),
        in_specs=[
            pl.BlockSpec((_NB, _D), lambda i: (i, 0)),
            pl.BlockSpec((_NB, 1), lambda i: (i, 0)),
        ],
        out_specs=pl.BlockSpec((_B, _D), lambda i: (0, 0)),
        out_shape=jax.ShapeDtypeStruct((_B, _D), jnp.float32),
    )(emb, bidx2d)


def _layer_kernel(emb_ref, bidx_ref, m_ref, blen_ref, wt_ref, b_ref, out_ref):
    emb = emb_ref[...]
    oh = _onehot(bidx_ref[...])
    m_g = jnp.dot(oh, m_ref[...], preferred_element_type=jnp.float32)
    denom = jnp.dot(oh, blen_ref[...], preferred_element_type=jnp.float32)
    mx = (m_g - emb) / (denom - 0.99999)
    wt = wt_ref[...]
    h = (jnp.dot(emb, wt[:_D], preferred_element_type=jnp.float32)
         + jnp.dot(mx, wt[_D:], preferred_element_type=jnp.float32)
         + b_ref[...])
    out_ref[...] = jnp.maximum(h, 0.0)


def _layer(emb, bidx2d, m, blen2d, wt, b2d):
    return pl.pallas_call(
        _layer_kernel,
        grid=(_N // _NB,),
        in_specs=[
            pl.BlockSpec((_NB, _D), lambda i: (i, 0)),
            pl.BlockSpec((_NB, 1), lambda i: (i, 0)),
            pl.BlockSpec((_B, _D), lambda i: (0, 0)),
            pl.BlockSpec((_B, 1), lambda i: (0, 0)),
            pl.BlockSpec((2 * _D, _D), lambda i: (0, 0)),
            pl.BlockSpec((1, _D), lambda i: (0, 0)),
        ],
        out_specs=pl.BlockSpec((_NB, _D), lambda i: (i, 0)),
        out_shape=jax.ShapeDtypeStruct((_N, _D), jnp.float32),
    )(emb, bidx2d, m, blen2d, wt, b2d)


def _head_kernel(m_ref, wht_ref, bh_ref, out_ref):
    out_ref[...] = (jnp.dot(m_ref[...], wht_ref[...],
                            preferred_element_type=jnp.float32)
                    + bh_ref[...])


def _head(m, wht, bh2d):
    return pl.pallas_call(
        _head_kernel,
        in_specs=[
            pl.BlockSpec((_B, _D), lambda: (0, 0)),
            pl.BlockSpec((_D, 1), lambda: (0, 0)),
            pl.BlockSpec((1, 1), lambda: (0, 0)),
        ],
        out_specs=pl.BlockSpec((_B, 1), lambda: (0, 0)),
        out_shape=jax.ShapeDtypeStruct((_B, 1), jnp.float32),
    )(m, wht, bh2d)


def kernel(x, item_starts, batch_idx, batch_len, emb_table, W0, b0, W1, b1, Wh, bh):
    t = x.shape[0]
    n = item_starts.shape[0]
    bag_id = jnp.searchsorted(item_starts, jnp.arange(t), side='right') - 1
    bag_id = jnp.clip(bag_id, 0, n - 1)
    emb_rows = jnp.take(emb_table, x, axis=0)
    sums = jax.ops.segment_sum(emb_rows, bag_id, num_segments=n)
    counts = jax.ops.segment_sum(jnp.ones((t,), jnp.float32), bag_id,
                                 num_segments=n)
    emb = jnp.where(counts[:, None] > 0,
                    sums / jnp.maximum(counts, 1.0)[:, None], 0.0)

    bidx2d = batch_idx.astype(jnp.int32).reshape(_N, 1)
    blen2d = batch_len.reshape(_B, 1)
    for (W, b) in ((W0, b0), (W1, b1)):
        m = _segsum(emb, bidx2d)
        emb = _layer(emb, bidx2d, m, blen2d, W.T, b.reshape(1, _D))
    m = _segsum(emb, bidx2d)
    return _head(m, Wh.T, bh.reshape(1, 1))


# speedup vs baseline: 26.1536x; 26.0540x over previous
import jax
import jax.numpy as jnp
from jax.experimental import pallas as pl

_N = 50000
_B = 2000
_D = 64
_NB = 1000


def _onehot(bidx_block):
    iota = jax.lax.broadcasted_iota(jnp.int32, (bidx_block.shape[0], _B), 1)
    return (bidx_block == iota).astype(jnp.float32)


def _segsum_kernel(emb_ref, bidx_ref, m_ref):
    step = pl.program_id(0)

    @pl.when(step == 0)
    def _():
        m_ref[...] = jnp.zeros_like(m_ref)

    oh = _onehot(bidx_ref[...])
    m_ref[...] += jax.lax.dot_general(
        oh, emb_ref[...], (((0,), (0,)), ((), ())),
        preferred_element_type=jnp.float32)


def _segsum(emb, bidx2d):
    return pl.pallas_call(
        _segsum_kernel,
        grid=(_N // _NB,),
        in_specs=[
            pl.BlockSpec((_NB, _D), lambda i: (i, 0)),
            pl.BlockSpec((_NB, 1), lambda i: (i, 0)),
        ],
        out_specs=pl.BlockSpec((_B, _D), lambda i: (0, 0)),
        out_shape=jax.ShapeDtypeStruct((_B, _D), jnp.float32),
    )(emb, bidx2d)


def _layer_kernel(emb_ref, bidx_ref, m_ref, blen_ref, wt_ref, b_ref, out_ref):
    emb = emb_ref[...]
    oh = _onehot(bidx_ref[...])
    m_g = jnp.dot(oh, m_ref[...], preferred_element_type=jnp.float32)
    denom = jnp.dot(oh, blen_ref[...], preferred_element_type=jnp.float32)
    mx = (m_g - emb) / (denom - 0.99999)
    wt = wt_ref[...]
    h = (jnp.dot(emb, wt[:_D], preferred_element_type=jnp.float32)
         + jnp.dot(mx, wt[_D:], preferred_element_type=jnp.float32)
         + b_ref[...])
    out_ref[...] = jnp.maximum(h, 0.0)


def _layer(emb, bidx2d, m, blen2d, wt, b2d):
    return pl.pallas_call(
        _layer_kernel,
        grid=(_N // _NB,),
        in_specs=[
            pl.BlockSpec((_NB, _D), lambda i: (i, 0)),
            pl.BlockSpec((_NB, 1), lambda i: (i, 0)),
            pl.BlockSpec((_B, _D), lambda i: (0, 0)),
            pl.BlockSpec((_B, 1), lambda i: (0, 0)),
            pl.BlockSpec((2 * _D, _D), lambda i: (0, 0)),
            pl.BlockSpec((1, _D), lambda i: (0, 0)),
        ],
        out_specs=pl.BlockSpec((_NB, _D), lambda i: (i, 0)),
        out_shape=jax.ShapeDtypeStruct((_N, _D), jnp.float32),
    )(emb, bidx2d, m, blen2d, wt, b2d)


def _head_kernel(m_ref, wht_ref, bh_ref, out_ref):
    out_ref[...] = (jnp.dot(m_ref[...], wht_ref[...],
                            preferred_element_type=jnp.float32)
                    + bh_ref[...])


def _head(m, wht, bh2d):
    return pl.pallas_call(
        _head_kernel,
        in_specs=[
            pl.BlockSpec((_B, _D), lambda: (0, 0)),
            pl.BlockSpec((_D, 1), lambda: (0, 0)),
            pl.BlockSpec((1, 1), lambda: (0, 0)),
        ],
        out_specs=pl.BlockSpec((_B, 1), lambda: (0, 0)),
        out_shape=jax.ShapeDtypeStruct((_B, 1), jnp.float32),
    )(m, wht, bh2d)


def kernel(x, item_starts, batch_idx, batch_len, emb_table, W0, b0, W1, b1, Wh, bh):
    t = x.shape[0]
    starts_ext = jnp.append(item_starts, t)
    counts = (starts_ext[1:] - starts_ext[:-1]).astype(jnp.float32)
    emb_rows = jnp.take(emb_table, x, axis=0)
    csum = jnp.concatenate(
        [jnp.zeros((1, _D), jnp.float32),
         jnp.cumsum(emb_rows, axis=0, dtype=jnp.float32)], axis=0)
    sums = jnp.take(csum, starts_ext[1:], axis=0) - jnp.take(
        csum, starts_ext[:-1], axis=0)
    emb = jnp.where(counts[:, None] > 0,
                    sums / jnp.maximum(counts, 1.0)[:, None], 0.0)

    bidx2d = batch_idx.astype(jnp.int32).reshape(_N, 1)
    blen2d = batch_len.reshape(_B, 1)
    for (W, b) in ((W0, b0), (W1, b1)):
        m = _segsum(emb, bidx2d)
        emb = _layer(emb, bidx2d, m, blen2d, W.T, b.reshape(1, _D))
    m = _segsum(emb, bidx2d)
    return _head(m, Wh.T, bh.reshape(1, 1))
